# Initial kernel scaffold; baseline (speedup 1.0000x reference)
import jax
import jax.numpy as jnp
from jax.experimental import pallas as pl


def kernel(pos, attr, params, edge_index, graph_ids):
    raise NotImplementedError("write your pallas kernel here")


# speedup vs baseline: 4.7910x; 4.7910x over previous
import functools

import jax
import jax.numpy as jnp
from jax import lax
from jax.experimental import pallas as pl
from jax.experimental.pallas import tpu as pltpu
from jax.experimental.pallas import tpu_sc as plsc

N = 50000
E = 800000
G = 2048
FH = 32

NS = 16
NC = 2

CH = 128
GRP = 16
NGRP = 25
NCH = GRP * NGRP
E_PAD = NS * NCH * CH
N_ACC = 51200
N_PAD = N_ACC
G_ACC = 2176

_mesh = plsc.VectorSubcoreMesh(
    core_axis_name="c", subcore_axis_name="s", num_cores=NC, num_subcores=NS)
_sc_params = pltpu.CompilerParams(use_tc_tiling_on_sc=False)


def _zero_rows(ref, nrows, width):
  def body(i, _):
    for j in range(width // 16):
      ref[i, pl.ds(j * 16, 16)] = jnp.zeros((16,), jnp.float32)
    return 0
  lax.fori_loop(0, nrows, body, 0, unroll=2)


def _fill_ones(ref, nrows, width):
  def body(i, _):
    for j in range(width // 16):
      ref[i, pl.ds(j * 16, 16)] = jnp.ones((16,), jnp.float32)
    return 0
  lax.fori_loop(0, nrows, body, 0, unroll=2)


@functools.partial(
    pl.kernel,
    out_type=(jax.ShapeDtypeStruct((N_ACC, 16), jnp.float32),
              jax.ShapeDtypeStruct((N_ACC, 16), jnp.float32)),
    mesh=_mesh,
    compiler_params=_sc_params,
    scratch_types=[
        pltpu.VMEM_SHARED((N_ACC, 16), jnp.float32),
        pltpu.VMEM((NCH // 2, CH), jnp.int32),
        pltpu.VMEM((CH, 16), jnp.float32),
        pltpu.VMEM((CH, 16), jnp.float32),
    ],
)
def _deg_sc(dstr, degA, degB, dacc, idx_v, ones_v, zrow):
  c = lax.axis_index("c")
  s = lax.axis_index("s")
  _zero_rows(zrow, CH, 16)
  _fill_ones(ones_v, CH, 16)
  base = s * (N_ACC // NS)
  for k in range(N_ACC // NS // CH):
    pltpu.sync_copy(zrow, dacc.at[pl.ds(base + k * CH, CH)])
  pltpu.sync_copy(dstr.at[s * NC + c], idx_v)
  plsc.subcore_barrier()

  def body(j, _):
    pltpu.sync_copy(ones_v, dacc.at[idx_v.at[j]], add=True)
    return 0
  lax.fori_loop(0, NCH // 2, body, 0)
  plsc.subcore_barrier()
  nrows = N_ACC // NS

  @pl.when(c == 0)
  def _():
    pltpu.sync_copy(dacc.at[pl.ds(base, nrows)], degA.at[pl.ds(base, nrows)])

  @pl.when(c == 1)
  def _():
    pltpu.sync_copy(dacc.at[pl.ds(base, nrows)], degB.at[pl.ds(base, nrows)])


@functools.partial(
    pl.kernel,
    out_type=(jax.ShapeDtypeStruct((N_ACC, FH), jnp.float32),
              jax.ShapeDtypeStruct((N_ACC, FH), jnp.float32)),
    mesh=_mesh,
    compiler_params=_sc_params,
    scratch_types=[
        pltpu.VMEM_SHARED((N_ACC, FH), jnp.float32),
        pltpu.VMEM((GRP, CH), jnp.int32),
        pltpu.VMEM((GRP, CH), jnp.int32),
        pltpu.VMEM((GRP, CH), jnp.int32),
        pltpu.VMEM((GRP, CH), jnp.int32),
        pltpu.VMEM((CH, FH), jnp.float32),
        pltpu.VMEM((CH, FH), jnp.float32),
        pltpu.SemaphoreType.DMA,
        pltpu.SemaphoreType.DMA,
        pltpu.SemaphoreType.DMA,
    ],
)
def _msum_sc(hA, hB, srcr, dstr, outA, outB,
             acc, src_g0, src_g1, dst_g0, dst_g1, rows0, rows1,
             sem0, sem1, isem):
  c = lax.axis_index("c")
  s = lax.axis_index("s")
  _zero_rows(rows0, CH, FH)
  base = s * (N_ACC // NS)
  for k in range(N_ACC // NS // CH):
    pltpu.sync_copy(rows0, acc.at[pl.ds(base + k * CH, CH)])
  plsc.subcore_barrier()
  sg = (src_g0, src_g1)
  dg = (dst_g0, dst_g1)

  def run(tbl):
    pltpu.async_copy(srcr.at[s, pl.ds(0, GRP)], sg[0], isem)
    pltpu.async_copy(dstr.at[s, pl.ds(0, GRP)], dg[0], isem)
    for g in range(NGRP):
      cs = sg[g % 2]
      cd = dg[g % 2]
      pltpu.make_async_copy(srcr.at[s, pl.ds(g * GRP, GRP)], cs, isem).wait()
      pltpu.make_async_copy(dstr.at[s, pl.ds(g * GRP, GRP)], cd, isem).wait()
      if g + 1 < NGRP:
        pltpu.async_copy(srcr.at[s, pl.ds((g + 1) * GRP, GRP)],
                         sg[(g + 1) % 2], isem)
        pltpu.async_copy(dstr.at[s, pl.ds((g + 1) * GRP, GRP)],
                         dg[(g + 1) % 2], isem)
      pltpu.async_copy(tbl.at[cs.at[0]], rows0, sem0)

      def body(i, _):
        j0 = 2 * i
        j1 = j0 + 1
        pltpu.async_copy(tbl.at[cs.at[j1]], rows1, sem1)
        pltpu.make_async_copy(tbl.at[cs.at[j0]], rows0, sem0).wait()
        pltpu.sync_copy(rows0, acc.at[cd.at[j0]], add=True)

        @pl.when(j0 + 2 < GRP)
        def _():
          pltpu.async_copy(tbl.at[cs.at[j0 + 2]], rows0, sem0)

        pltpu.make_async_copy(tbl.at[cs.at[j1]], rows1, sem1).wait()
        pltpu.sync_copy(rows1, acc.at[cd.at[j1]], add=True)
        return 0

      lax.fori_loop(0, GRP // 2, body, 0)

  @pl.when(c == 0)
  def _():
    run(hA)

  @pl.when(c == 1)
  def _():
    run(hB)

  plsc.subcore_barrier()
  nrows = N_ACC // NS

  @pl.when(c == 0)
  def _():
    pltpu.sync_copy(acc.at[pl.ds(base, nrows)], outA.at[pl.ds(base, nrows)])

  @pl.when(c == 1)
  def _():
    pltpu.sync_copy(acc.at[pl.ds(base, nrows)], outB.at[pl.ds(base, nrows)])


@functools.partial(
    pl.kernel,
    out_type=(jax.ShapeDtypeStruct((G_ACC, FH), jnp.float32),
              jax.ShapeDtypeStruct((G_ACC, FH), jnp.float32)),
    mesh=_mesh,
    compiler_params=_sc_params,
    scratch_types=[
        pltpu.VMEM_SHARED((G_ACC, FH), jnp.float32),
        pltpu.VMEM((N_PAD // NS // CH, CH), jnp.int32),
        pltpu.VMEM((CH, FH), jnp.float32),
        pltpu.VMEM((G_ACC // NS, FH), jnp.float32),
    ],
)
def _readout_sc(hA, hB, gidr, gA, gB, gacc, idx_v, rows, zrow):
  c = lax.axis_index("c")
  s = lax.axis_index("s")
  grows = G_ACC // NS
  _zero_rows(zrow, grows, FH)
  pltpu.sync_copy(zrow, gacc.at[pl.ds(s * grows, grows)])
  pltpu.sync_copy(gidr.at[s], idx_v)
  plsc.subcore_barrier()
  nchunk = N_PAD // NS // CH
  base = s * (N_PAD // NS)

  def run(tbl):
    def body(j, _):
      pltpu.sync_copy(tbl.at[pl.ds(base + j * CH, CH)], rows)
      pltpu.sync_copy(rows, gacc.at[idx_v.at[j]], add=True)
      return 0
    lax.fori_loop(0, nchunk, body, 0)

  @pl.when(c == 0)
  def _():
    run(hA)

  @pl.when(c == 1)
  def _():
    run(hB)

  plsc.subcore_barrier()

  @pl.when(c == 0)
  def _():
    pltpu.sync_copy(gacc.at[pl.ds(s * grows, grows)], gA.at[pl.ds(s * grows, grows)])

  @pl.when(c == 1)
  def _():
    pltpu.sync_copy(gacc.at[pl.ds(s * grows, grows)], gB.at[pl.ds(s * grows, grows)])


def _dot(a, b):
  return jax.lax.dot_general(a, b, (((1,), (0,)), ((), ())),
                             preferred_element_type=jnp.float32)


def _enc_body(x, w1, b1, w2, b2, w3, b3, oA, oB):
  h = jnp.maximum(_dot(x[...], w1[...]) + b1[...], 0.0)
  h = jnp.maximum(_dot(h, w2[...]) + b2[...], 0.0)
  h = _dot(h, w3[...]) + b3[...]
  oA[...] = h[:, :FH]
  oB[...] = h[:, FH:]


def _fcn_body(hA, hB, mA, mB, inv, w1, b1, w2, b2, w3, b3, w4, b4, oA, oB):
  iv = inv[...]
  x = jnp.concatenate([hA[...], hB[...], mA[...] * iv, mB[...] * iv], axis=1)
  t = jnp.maximum(_dot(x, w1[...]) + b1[...], 0.0)
  t = jnp.maximum(_dot(t, w2[...]) + b2[...], 0.0)
  t = jnp.maximum(_dot(t, w3[...]) + b3[...], 0.0)
  t = _dot(t, w4[...]) + b4[...]
  oA[...] = t[:, :FH]
  oB[...] = t[:, FH:]


def _pred_body(gA, gB, w1, b1, w2, b2, w3, b3, w4, b4, w5, b5, out):
  g = jnp.concatenate([gA[...], gB[...]], axis=1)
  t = jnp.maximum(_dot(g, w1[...]) + b1[...], 0.0)
  t = jnp.maximum(_dot(t, w2[...]) + b2[...], 0.0)
  t = jnp.maximum(_dot(t, w3[...]) + b3[...], 0.0)
  t = jnp.maximum(_dot(t, w4[...]) + b4[...], 0.0)
  out[...] = _dot(t, w5[...]) + b5[...]


def _full_spec(shape):
  return pl.BlockSpec(shape, lambda i: tuple(0 for _ in shape))


def _rows_spec(rows, width):
  return pl.BlockSpec((rows, width), lambda i: (i, 0))


def _wspecs(layers):
  specs = []
  for l in layers:
    specs.append(_full_spec(l["W"].shape))
    specs.append(_full_spec((1, l["b"].shape[0])))
  return specs


def _wargs(layers):
  args = []
  for l in layers:
    args.append(l["W"])
    args.append(l["b"].reshape(1, -1))
  return args


_BLK = 1000


def kernel(pos, attr, params, edge_index, graph_ids):
  x = jnp.concatenate([pos, attr], axis=-1)
  src = edge_index[0]
  dst = edge_index[1]
  srcr = jnp.concatenate(
      [src, jnp.zeros((E_PAD - E,), jnp.int32)]).reshape(NS, NCH, CH)
  dst_p = jnp.concatenate([dst, jnp.full((E_PAD - E,), N, jnp.int32)])
  dstr = dst_p.reshape(NS, NCH, CH)
  dstr2 = dst_p.reshape(NS * NC, NCH // 2, CH)
  gidr = jnp.concatenate(
      [graph_ids, jnp.full((N_PAD - N,), G, jnp.int32)]).reshape(
          NS, N_PAD // NS // CH, CH)

  enc = params["enc"]
  hA, hB = pl.pallas_call(
      _enc_body,
      grid=(N // _BLK,),
      in_specs=[_rows_spec(_BLK, 4)] + _wspecs(enc),
      out_specs=(_rows_spec(_BLK, FH), _rows_spec(_BLK, FH)),
      out_shape=(jax.ShapeDtypeStruct((N, FH), jnp.float32),
                 jax.ShapeDtypeStruct((N, FH), jnp.float32)),
  )(x, *_wargs(enc))

  degA, degB = _deg_sc(dstr2)
  deg = degA[:N, 0] + degB[:N, 0]
  inv = (1.0 / jnp.maximum(deg, 1.0)).reshape(N, 1)

  for net in params["node_nets"]:
    mA, mB = _msum_sc(hA, hB, srcr, dstr)
    hA, hB = pl.pallas_call(
        _fcn_body,
        grid=(N // _BLK,),
        in_specs=[_rows_spec(_BLK, FH)] * 2 + [_rows_spec(_BLK, FH)] * 2
                 + [_rows_spec(_BLK, 1)] + _wspecs(net),
        out_specs=(_rows_spec(_BLK, FH), _rows_spec(_BLK, FH)),
        out_shape=(jax.ShapeDtypeStruct((N, FH), jnp.float32),
                   jax.ShapeDtypeStruct((N, FH), jnp.float32)),
    )(hA, hB, mA, mB, inv, *_wargs(net))

  hA_p = jnp.pad(hA, ((0, N_PAD - N), (0, 0)))
  hB_p = jnp.pad(hB, ((0, N_PAD - N), (0, 0)))
  gA, gB = _readout_sc(hA_p, hB_p, gidr)

  pred = params["pred"]
  out = pl.pallas_call(
      _pred_body,
      grid=(G // 512,),
      in_specs=[_rows_spec(512, FH)] * 2 + _wspecs(pred),
      out_specs=_rows_spec(512, 1),
      out_shape=jax.ShapeDtypeStruct((G, 1), jnp.float32),
  )(gA, gB, *_wargs(pred))
  return out
